# baseline (speedup 1.0000x reference)
import jax
import jax.numpy as jnp
from jax.experimental import pallas as pl
from jax.experimental.pallas import tpu as pltpu

N_VARS = 64
EMBED_DIM = 64
_NSEM = 8


def _fill(emb_t, scratch_ref, n_patches):
    for v in range(N_VARS):
        scratch_ref[pl.ds(v * EMBED_DIM, EMBED_DIM), :] = jnp.broadcast_to(
            emb_t[:, v : v + 1], (EMBED_DIM, n_patches)
        )


def _bcast_kernel(emb_ref, out_ref, scratch_a, scratch_b, sems):
    emb_t = jnp.transpose(emb_ref[...], (1, 0))
    n_patches = out_ref.shape[1]
    _fill(emb_t, scratch_a, n_patches)
    _fill(emb_t, scratch_b, n_patches)
    B = out_ref.shape[0] // scratch_a.shape[0]
    flat = scratch_a.shape[0]
    srcs = (scratch_a, scratch_b)
    for b in range(B):
        pltpu.make_async_copy(
            srcs[b % 2],
            out_ref.at[pl.ds(b * flat, flat), :],
            sems.at[b % _NSEM],
        ).start()
    for b in range(B):
        pltpu.make_async_copy(
            srcs[b % 2],
            out_ref.at[pl.ds(b * flat, flat), :],
            sems.at[b % _NSEM],
        ).wait()


def kernel(x, channel_emb):
    B, n_patches, _ = x.shape
    flat = N_VARS * EMBED_DIM
    out2d = pl.pallas_call(
        _bcast_kernel,
        in_specs=[pl.BlockSpec(memory_space=pltpu.VMEM)],
        out_specs=pl.BlockSpec(memory_space=pl.ANY),
        out_shape=jax.ShapeDtypeStruct((B * flat, n_patches), channel_emb.dtype),
        scratch_shapes=[
            pltpu.VMEM((flat, n_patches), channel_emb.dtype),
            pltpu.VMEM((flat, n_patches), channel_emb.dtype),
            pltpu.SemaphoreType.DMA((_NSEM,)),
        ],
    )(channel_emb)
    out_t = out2d.reshape(B, N_VARS, EMBED_DIM, n_patches)
    return out_t.transpose(0, 3, 1, 2)


# speedup vs baseline: 1.0021x; 1.0021x over previous
import jax
import jax.numpy as jnp
from jax.experimental import pallas as pl
from jax.experimental.pallas import tpu as pltpu

N_VARS = 64
EMBED_DIM = 64
_NSEM = 8


def _fill(emb_t, scratch_ref, n_patches):
    for v in range(N_VARS):
        scratch_ref[pl.ds(v * EMBED_DIM, EMBED_DIM), :] = jnp.broadcast_to(
            emb_t[:, v : v + 1], (EMBED_DIM, n_patches)
        )


def _bcast_kernel(emb_ref, out_ref, scratch_a, scratch_b, sems):
    emb_t = jnp.transpose(emb_ref[...], (1, 0))
    n_patches = out_ref.shape[1]
    B = out_ref.shape[0] // scratch_a.shape[0]
    flat = scratch_a.shape[0]
    srcs = (scratch_a, scratch_b)
    _fill(emb_t, scratch_a, n_patches)
    for b in range(0, B, 2):
        pltpu.make_async_copy(
            scratch_a,
            out_ref.at[pl.ds(b * flat, flat), :],
            sems.at[b % _NSEM],
        ).start()
    _fill(emb_t, scratch_b, n_patches)
    for b in range(1, B, 2):
        pltpu.make_async_copy(
            scratch_b,
            out_ref.at[pl.ds(b * flat, flat), :],
            sems.at[b % _NSEM],
        ).start()
    for b in range(B):
        pltpu.make_async_copy(
            srcs[b % 2],
            out_ref.at[pl.ds(b * flat, flat), :],
            sems.at[b % _NSEM],
        ).wait()


def kernel(x, channel_emb):
    B, n_patches, _ = x.shape
    flat = N_VARS * EMBED_DIM
    out2d = pl.pallas_call(
        _bcast_kernel,
        in_specs=[pl.BlockSpec(memory_space=pltpu.VMEM)],
        out_specs=pl.BlockSpec(memory_space=pl.ANY),
        out_shape=jax.ShapeDtypeStruct((B * flat, n_patches), channel_emb.dtype),
        scratch_shapes=[
            pltpu.VMEM((flat, n_patches), channel_emb.dtype),
            pltpu.VMEM((flat, n_patches), channel_emb.dtype),
            pltpu.SemaphoreType.DMA((_NSEM,)),
        ],
    )(channel_emb)
    out_t = out2d.reshape(B, N_VARS, EMBED_DIM, n_patches)
    return out_t.transpose(0, 3, 1, 2)
